# Initial kernel scaffold; baseline (speedup 1.0000x reference)
import jax
import jax.numpy as jnp
from jax.experimental import pallas as pl


def kernel(inputs, codebook, bias):
    raise NotImplementedError("write your pallas kernel here")


# speedup vs baseline: 11.0449x; 11.0449x over previous
import functools

import jax
import jax.numpy as jnp
from jax.experimental import pallas as pl

_N = 1024
_D = 32
_OPTIONS = 512
_NUM_STAGES = 4
_BLOCK_N = 256


def _encoder_body(x_ref, cb_ref, bias_ref, enc_ref, cur_ref, loss_ref):
    x = x_ref[...]
    bias = bias_ref[...]
    current = jnp.zeros_like(x)
    idxs = []
    for i in range(_NUM_STAGES):
        cb = cb_ref[i]
        if i == 0:
            cb = cb + bias
        r = current - x
        dot = jax.lax.dot_general(
            r, cb, (((1,), (1,)), ((), ())),
            preferred_element_type=jnp.float32,
            precision=jax.lax.Precision.HIGHEST,
        )
        sq_r = jnp.sum(r * r, axis=1, keepdims=True)
        sq_c = jnp.sum(cb * cb, axis=1)[None, :]
        loss = (sq_r + 2.0 * dot + sq_c) * (1.0 / _D)
        loss_ref[:, i, :] = loss
        idx = jnp.argmin(loss, axis=1)
        idxs.append(idx)
        onehot = (
            jax.lax.broadcasted_iota(jnp.int32, loss.shape, 1) == idx[:, None]
        ).astype(jnp.float32)
        chosen = jax.lax.dot_general(
            onehot, cb, (((1,), (0,)), ((), ())),
            preferred_element_type=jnp.float32,
            precision=jax.lax.Precision.HIGHEST,
        )
        current = current + chosen
    enc_ref[...] = jnp.stack(idxs, axis=1)
    cur_ref[...] = current


@jax.jit
def kernel(inputs, codebook, bias):
    n, d = inputs.shape
    num_stages, options, _ = codebook.shape
    grid = (n // _BLOCK_N,)
    enc, current, losses = pl.pallas_call(
        _encoder_body,
        grid=grid,
        in_specs=[
            pl.BlockSpec((_BLOCK_N, d), lambda j: (j, 0)),
            pl.BlockSpec((num_stages, options, d), lambda j: (0, 0, 0)),
            pl.BlockSpec((options, d), lambda j: (0, 0)),
        ],
        out_specs=[
            pl.BlockSpec((_BLOCK_N, num_stages), lambda j: (j, 0)),
            pl.BlockSpec((_BLOCK_N, d), lambda j: (j, 0)),
            pl.BlockSpec((_BLOCK_N, num_stages, options), lambda j: (j, 0, 0)),
        ],
        out_shape=[
            jax.ShapeDtypeStruct((n, num_stages), jnp.int32),
            jax.ShapeDtypeStruct((n, d), jnp.float32),
            jax.ShapeDtypeStruct((n, num_stages, options), jnp.float32),
        ],
    )(inputs, codebook, bias)
    return enc, current, losses
